# baseline (speedup 1.0000x reference)
import jax
import jax.numpy as jnp
from jax import lax
from jax.experimental import pallas as pl
from jax.experimental.pallas import tpu as pltpu
from jax.experimental.pallas import tpu_sc as plsc

N_TOK = 16384
N_CLUSTERS = 8192
DIM = 32
TOK_TILE = 512
GATHER_WINDOW = 128


ROW_TILE = 64
LANES = 128
N_CHUNKS = N_CLUSTERS // LANES


def _argmin_body(x_ref, b_ref, idx_ref):
    a = x_ref[...]
    t = lax.dot_general(
        a, b_ref[...], (((1,), (1,)), ((), ())),
        preferred_element_type=jnp.float32,
        precision=lax.Precision.DEFAULT,
    )
    lane_iota = lax.broadcasted_iota(jnp.int32, (ROW_TILE, LANES), 1)
    for r in range(TOK_TILE // ROW_TILE):
        rows = slice(r * ROW_TILE, (r + 1) * ROW_TILE)
        bestv = t[rows, 0:LANES]
        bestj = jnp.zeros((ROW_TILE, LANES), jnp.int32)
        for j in range(1, N_CHUNKS):
            v = t[rows, j * LANES:(j + 1) * LANES]
            lt = v < bestv
            bestv = jnp.where(lt, v, bestv)
            bestj = jnp.where(lt, jnp.int32(j), bestj)
        rowmin = jnp.min(bestv, axis=1, keepdims=True)
        k = bestj * LANES + lane_iota
        cand = jnp.where(bestv == rowmin, k, jnp.int32(N_CLUSTERS * 2))
        idx_ref[rows, :] = jnp.min(cand, axis=1, keepdims=True)


def _nearest_indices(a_aug, b_mat):
    n = a_aug.shape[0]
    return pl.pallas_call(
        _argmin_body,
        grid=(n // TOK_TILE,),
        in_specs=[
            pl.BlockSpec((TOK_TILE, DIM + 8), lambda i: (i, 0)),
            pl.BlockSpec((N_CLUSTERS, DIM + 8), lambda i: (0, 0)),
        ],
        out_specs=pl.BlockSpec((TOK_TILE, 1), lambda i: (i, 0)),
        out_shape=jax.ShapeDtypeStruct((n, 1), jnp.int32),
    )(a_aug, b_mat)


def _sc_gather(table, indices_row):
    n = indices_row.shape[1]

    @pl.kernel(
        out_type=jax.ShapeDtypeStruct((n, DIM), table.dtype),
        mesh=plsc.VectorSubcoreMesh(
            core_axis_name="core", subcore_axis_name="subcore"
        ),
        scratch_types=[pltpu.VMEM((GATHER_WINDOW, 128), jnp.float32)],
    )
    def gather_kernel(tab_hbm, i_hbm, o_hbm, scratch):
        def body(i_vmem, o_vmem):
            pltpu.sync_copy(tab_hbm.at[i_vmem.at[0]], scratch)

            @pl.loop(0, GATHER_WINDOW)
            def _(r):
                @pl.loop(0, DIM, step=16)
                def _(c):
                    slc = (pl.ds(r, 1), pl.ds(c, 16))
                    o_vmem.at[*slc][...] = scratch.at[*slc][...]

        pltpu.emit_pipeline(
            body,
            grid=(n // GATHER_WINDOW,),
            in_specs=[
                pl.BlockSpec((1, GATHER_WINDOW), index_map=lambda i: (0, i))
            ],
            out_specs=[
                pl.BlockSpec((GATHER_WINDOW, DIM), index_map=lambda i: (i, 0))
            ],
            core_axis_name=("core", "subcore"),
            dimension_semantics=(pltpu.PARALLEL,),
        )(i_hbm, o_hbm)

    return gather_kernel(table, indices_row)


def kernel(x, cluster_centers):
    b, t, d = x.shape
    x_flat = x.reshape(b * t, d)
    c2 = jnp.sum(cluster_centers * cluster_centers, axis=-1, keepdims=True)
    hi = lax.reduce_precision(c2, 8, 7)
    r1 = c2 - hi
    mid = lax.reduce_precision(r1, 8, 7)
    lo = lax.reduce_precision(r1 - mid, 8, 7)
    b_mat = jnp.concatenate(
        [-2.0 * cluster_centers, jnp.ones((N_CLUSTERS, 1), jnp.float32),
         hi, mid, lo, jnp.zeros((N_CLUSTERS, 4), jnp.float32)], axis=1)
    xs = x_flat + 1e-06
    x2_col = jnp.sum(xs * xs, axis=1, keepdims=True)
    a_aug = jnp.concatenate(
        [xs, x2_col, jnp.ones((N_TOK, 3), jnp.float32),
         jnp.zeros((N_TOK, 4), jnp.float32)], axis=1)
    table_pad = jnp.pad(cluster_centers, ((0, 0), (0, 128 - DIM)))
    half = N_TOK // 2
    outs = []
    for h in range(2):
        a_h = a_aug[h * half:(h + 1) * half]
        idx_h = _nearest_indices(a_h, b_mat)
        outs.append(_sc_gather(table_pad, idx_h.reshape(1, half)))
    out = jnp.concatenate(outs, axis=0)
    return out.reshape(b, t, d)


# speedup vs baseline: 1.0466x; 1.0466x over previous
import jax
import jax.numpy as jnp
from jax import lax
from jax.experimental import pallas as pl
from jax.experimental.pallas import tpu as pltpu
from jax.experimental.pallas import tpu_sc as plsc

N_TOK = 16384
N_CLUSTERS = 8192
DIM = 32
TOK_TILE = 1024
GATHER_WINDOW = 128


ROW_TILE = 64
LANES = 128
N_CHUNKS = N_CLUSTERS // LANES


def _argmin_body(x_ref, b_ref, idx_ref):
    a = x_ref[...]
    t = lax.dot_general(
        a, b_ref[...], (((1,), (1,)), ((), ())),
        preferred_element_type=jnp.float32,
        precision=lax.Precision.DEFAULT,
    )
    lane_iota = lax.broadcasted_iota(jnp.int32, (ROW_TILE, LANES), 1)
    for r in range(TOK_TILE // ROW_TILE):
        rows = slice(r * ROW_TILE, (r + 1) * ROW_TILE)
        bestv = t[rows, 0:LANES]
        bestj = jnp.zeros((ROW_TILE, LANES), jnp.int32)
        for j in range(1, N_CHUNKS):
            v = t[rows, j * LANES:(j + 1) * LANES]
            lt = v < bestv
            bestv = jnp.where(lt, v, bestv)
            bestj = jnp.where(lt, jnp.int32(j), bestj)
        rowmin = jnp.min(bestv, axis=1, keepdims=True)
        k = bestj * LANES + lane_iota
        cand = jnp.where(bestv == rowmin, k, jnp.int32(N_CLUSTERS * 2))
        idx_ref[rows, :] = jnp.min(cand, axis=1, keepdims=True)


def _nearest_indices(a_aug, b_mat):
    n = a_aug.shape[0]
    return pl.pallas_call(
        _argmin_body,
        grid=(n // TOK_TILE,),
        in_specs=[
            pl.BlockSpec((TOK_TILE, DIM + 8), lambda i: (i, 0)),
            pl.BlockSpec((N_CLUSTERS, DIM + 8), lambda i: (0, 0)),
        ],
        out_specs=pl.BlockSpec((TOK_TILE, 1), lambda i: (i, 0)),
        out_shape=jax.ShapeDtypeStruct((n, 1), jnp.int32),
    )(a_aug, b_mat)


def _sc_gather(table, indices_row):
    n = indices_row.shape[1]

    @pl.kernel(
        out_type=jax.ShapeDtypeStruct((n, DIM), table.dtype),
        mesh=plsc.VectorSubcoreMesh(
            core_axis_name="core", subcore_axis_name="subcore"
        ),
        scratch_types=[pltpu.VMEM((GATHER_WINDOW, 128), jnp.float32)],
    )
    def gather_kernel(tab_hbm, i_hbm, o_hbm, scratch):
        def body(i_vmem, o_vmem):
            pltpu.sync_copy(tab_hbm.at[i_vmem.at[0]], scratch)

            @pl.loop(0, GATHER_WINDOW)
            def _(r):
                @pl.loop(0, DIM, step=16)
                def _(c):
                    slc = (pl.ds(r, 1), pl.ds(c, 16))
                    o_vmem.at[*slc][...] = scratch.at[*slc][...]

        pltpu.emit_pipeline(
            body,
            grid=(n // GATHER_WINDOW,),
            in_specs=[
                pl.BlockSpec((1, GATHER_WINDOW), index_map=lambda i: (0, i))
            ],
            out_specs=[
                pl.BlockSpec((GATHER_WINDOW, DIM), index_map=lambda i: (i, 0))
            ],
            core_axis_name=("core", "subcore"),
            dimension_semantics=(pltpu.PARALLEL,),
        )(i_hbm, o_hbm)

    return gather_kernel(table, indices_row)


def kernel(x, cluster_centers):
    b, t, d = x.shape
    x_flat = x.reshape(b * t, d)
    c2 = jnp.sum(cluster_centers * cluster_centers, axis=-1, keepdims=True)
    hi = lax.reduce_precision(c2, 8, 7)
    r1 = c2 - hi
    mid = lax.reduce_precision(r1, 8, 7)
    lo = lax.reduce_precision(r1 - mid, 8, 7)
    b_mat = jnp.concatenate(
        [-2.0 * cluster_centers, jnp.ones((N_CLUSTERS, 1), jnp.float32),
         hi, mid, lo, jnp.zeros((N_CLUSTERS, 4), jnp.float32)], axis=1)
    xs = x_flat + 1e-06
    x2_col = jnp.sum(xs * xs, axis=1, keepdims=True)
    a_aug = jnp.concatenate(
        [xs, x2_col, jnp.ones((N_TOK, 3), jnp.float32),
         jnp.zeros((N_TOK, 4), jnp.float32)], axis=1)
    table_pad = jnp.pad(cluster_centers, ((0, 0), (0, 128 - DIM)))
    idx = _nearest_indices(a_aug, b_mat)
    out = _sc_gather(table_pad, idx.reshape(1, N_TOK))
    return out.reshape(b, t, d)


# speedup vs baseline: 1.0552x; 1.0083x over previous
import jax
import jax.numpy as jnp
from jax import lax
from jax.experimental import pallas as pl
from jax.experimental.pallas import tpu as pltpu
from jax.experimental.pallas import tpu_sc as plsc

N_TOK = 16384
N_CLUSTERS = 8192
DIM = 32
TOK_TILE = 1024
GATHER_WINDOW = 256


ROW_TILE = 64
LANES = 128
N_CHUNKS = N_CLUSTERS // LANES


def _argmin_body(x_ref, b_ref, idx_ref):
    a = x_ref[...]
    t = lax.dot_general(
        a, b_ref[...], (((1,), (1,)), ((), ())),
        preferred_element_type=jnp.float32,
        precision=lax.Precision.DEFAULT,
    )
    lane_iota = lax.broadcasted_iota(jnp.int32, (ROW_TILE, LANES), 1)
    for r in range(TOK_TILE // ROW_TILE):
        rows = slice(r * ROW_TILE, (r + 1) * ROW_TILE)
        bestv = t[rows, 0:LANES]
        bestj = jnp.zeros((ROW_TILE, LANES), jnp.int32)
        for j in range(1, N_CHUNKS):
            v = t[rows, j * LANES:(j + 1) * LANES]
            lt = v < bestv
            bestv = jnp.where(lt, v, bestv)
            bestj = jnp.where(lt, jnp.int32(j), bestj)
        rowmin = jnp.min(bestv, axis=1, keepdims=True)
        k = bestj * LANES + lane_iota
        cand = jnp.where(bestv == rowmin, k, jnp.int32(N_CLUSTERS * 2))
        idx_ref[rows, :] = jnp.min(cand, axis=1, keepdims=True)


def _nearest_indices(a_aug, b_mat):
    n = a_aug.shape[0]
    return pl.pallas_call(
        _argmin_body,
        grid=(n // TOK_TILE,),
        in_specs=[
            pl.BlockSpec((TOK_TILE, DIM + 8), lambda i: (i, 0)),
            pl.BlockSpec((N_CLUSTERS, DIM + 8), lambda i: (0, 0)),
        ],
        out_specs=pl.BlockSpec((TOK_TILE, 1), lambda i: (i, 0)),
        out_shape=jax.ShapeDtypeStruct((n, 1), jnp.int32),
    )(a_aug, b_mat)


def _sc_gather(table, indices_row):
    n = indices_row.shape[1]

    @pl.kernel(
        out_type=jax.ShapeDtypeStruct((n, 128), table.dtype),
        mesh=plsc.VectorSubcoreMesh(
            core_axis_name="core", subcore_axis_name="subcore"
        ),
    )
    def gather_kernel(tab_hbm, i_hbm, o_hbm):
        def body(i_vmem, o_vmem):
            pltpu.sync_copy(tab_hbm.at[i_vmem.at[0]], o_vmem)

        pltpu.emit_pipeline(
            body,
            grid=(n // GATHER_WINDOW,),
            in_specs=[
                pl.BlockSpec((1, GATHER_WINDOW), index_map=lambda i: (0, i))
            ],
            out_specs=[
                pl.BlockSpec((GATHER_WINDOW, 128), index_map=lambda i: (i, 0))
            ],
            core_axis_name=("core", "subcore"),
            dimension_semantics=(pltpu.PARALLEL,),
        )(i_hbm, o_hbm)

    return gather_kernel(table, indices_row)


def kernel(x, cluster_centers):
    b, t, d = x.shape
    x_flat = x.reshape(b * t, d)
    c2 = jnp.sum(cluster_centers * cluster_centers, axis=-1, keepdims=True)
    hi = lax.reduce_precision(c2, 8, 7)
    r1 = c2 - hi
    mid = lax.reduce_precision(r1, 8, 7)
    lo = lax.reduce_precision(r1 - mid, 8, 7)
    b_mat = jnp.concatenate(
        [-2.0 * cluster_centers, jnp.ones((N_CLUSTERS, 1), jnp.float32),
         hi, mid, lo, jnp.zeros((N_CLUSTERS, 4), jnp.float32)], axis=1)
    xs = x_flat + 1e-06
    x2_col = jnp.sum(xs * xs, axis=1, keepdims=True)
    a_aug = jnp.concatenate(
        [xs, x2_col, jnp.ones((N_TOK, 3), jnp.float32),
         jnp.zeros((N_TOK, 4), jnp.float32)], axis=1)
    table_pad = jnp.pad(cluster_centers, ((0, 0), (0, 128 - DIM)))
    idx = _nearest_indices(a_aug, b_mat)
    out = _sc_gather(table_pad, idx.reshape(1, N_TOK))
    return out[:, :DIM].reshape(b, t, d)
